# baseline (speedup 1.0000x reference)
import functools

import jax
import jax.numpy as jnp
from jax import lax
from jax.experimental import pallas as pl
from jax.experimental.pallas import tpu as pltpu
from jax.experimental.pallas import tpu_sc as plsc

BATCH = 16384
EMB = 128
LANES = 16
GROUPS = EMB // LANES
CHUNK = 128
N_COND = 8

SC_ROWS = 8192
TC_BLK = 1024


def _sc_half(embedded_x, c, masks):
    info = plsc.get_sparse_core_info()
    n_workers = info.num_cores * info.num_subcores
    b_per_w = SC_ROWS // n_workers
    n_chunks = b_per_w // CHUNK
    grp_per_chunk = CHUNK // LANES

    mesh = plsc.VectorSubcoreMesh(core_axis_name="c", subcore_axis_name="s")

    @functools.partial(
        pl.kernel,
        mesh=mesh,
        out_type=jax.ShapeDtypeStruct((SC_ROWS, EMB), jnp.float32),
        scratch_types=[
            pltpu.VMEM((b_per_w,), jnp.int32),
            pltpu.VMEM((N_COND, EMB), jnp.float32),
            pltpu.VMEM((b_per_w, EMB), jnp.float32),
        ]
        + [pltpu.SemaphoreType.DMA for _ in range(n_chunks + 3)],
    )
    def run(x_hbm, c_hbm, m_hbm, out_hbm, idx_v, masks_v, xbuf, *sems):
        load_sems = sems[:n_chunks]
        store_sem, idx_sem, msk_sem = sems[n_chunks:]

        wid = lax.axis_index("s") * info.num_cores + lax.axis_index("c")
        base = wid * b_per_w

        idx_cp = pltpu.async_copy(
            c_hbm.at[pl.ds(base, b_per_w)], idx_v, idx_sem)
        msk_cp = pltpu.async_copy(m_hbm, masks_v, msk_sem)
        loads = [
            pltpu.async_copy(
                x_hbm.at[pl.ds(base + j * CHUNK, CHUNK)],
                xbuf.at[pl.ds(j * CHUNK, CHUNK)],
                load_sems[j])
            for j in range(n_chunks)
        ]
        idx_cp.wait()
        msk_cp.wait()

        stores = [
            pltpu.make_async_copy(
                xbuf.at[pl.ds(j * CHUNK, CHUNK)],
                out_hbm.at[pl.ds(base + j * CHUNK, CHUNK)],
                store_sem)
            for j in range(n_chunks)
        ]

        def grp_body(t, carry):
            for j in range(n_chunks):

                @pl.when(t == j * grp_per_chunk)
                def _(j=j):
                    loads[j].wait()
                    if j > 0:
                        stores[j - 1].start()

            cvec = idx_v[pl.ds(t * LANES, LANES)]
            for l in range(LANES):
                r = t * LANES + l
                rowc = cvec[l]
                prods = []
                for g in range(GROUPS):
                    sl = pl.ds(g * LANES, LANES)
                    prods.append(xbuf[r, sl] * masks_v[rowc, sl])
                for g in range(GROUPS):
                    xbuf[r, pl.ds(g * LANES, LANES)] = prods[g]
            return carry

        lax.fori_loop(0, n_chunks * grp_per_chunk, grp_body, 0)
        stores[n_chunks - 1].start()
        for s in stores:
            s.wait()

    return run(embedded_x, c, masks)


def _tc_body(c_ref, m_ref, x_ref, o_ref):
    cb = c_ref[0, 0, :]
    onehot = (cb[:, None] == jax.lax.broadcasted_iota(
        jnp.int32, (1, N_COND), 1)).astype(jnp.float32)
    m = jnp.dot(onehot, m_ref[...], preferred_element_type=jnp.float32)
    o_ref[...] = x_ref[...] * m


def _tc_half(embedded_x, c, masks):
    rows = BATCH - SC_ROWS
    nb = rows // TC_BLK
    c2 = c.reshape(nb, 1, TC_BLK)
    return pl.pallas_call(
        _tc_body,
        grid=(nb,),
        in_specs=[
            pl.BlockSpec((1, 1, TC_BLK), lambda i: (i, 0, 0)),
            pl.BlockSpec((N_COND, EMB), lambda i: (0, 0)),
            pl.BlockSpec((TC_BLK, EMB), lambda i: (i, 0)),
        ],
        out_specs=pl.BlockSpec((TC_BLK, EMB), lambda i: (i, 0)),
        out_shape=jax.ShapeDtypeStruct((rows, EMB), jnp.float32),
    )(c2, masks, embedded_x)


def kernel(embedded_x, c, masks):
    c = c.astype(jnp.int32)
    out_sc = _sc_half(embedded_x[:SC_ROWS], c[:SC_ROWS], masks)
    out_tc = _tc_half(embedded_x[SC_ROWS:], c[SC_ROWS:], masks)
    return jnp.concatenate([out_sc, out_tc], axis=0)


# speedup vs baseline: 1.4397x; 1.4397x over previous
import functools

import jax
import jax.numpy as jnp
from jax import lax
from jax.experimental import pallas as pl
from jax.experimental.pallas import tpu as pltpu
from jax.experimental.pallas import tpu_sc as plsc

BATCH = 16384
EMB = 128
LANES = 16
GROUPS = EMB // LANES
CHUNK = 64
N_COND = 8


def kernel(embedded_x, c, masks):
    info = plsc.get_sparse_core_info()
    n_workers = info.num_cores * info.num_subcores
    b_per_w = BATCH // n_workers
    n_chunks = b_per_w // CHUNK
    grp_per_chunk = CHUNK // LANES

    mesh = plsc.VectorSubcoreMesh(core_axis_name="c", subcore_axis_name="s")

    @functools.partial(
        pl.kernel,
        mesh=mesh,
        out_type=jax.ShapeDtypeStruct((BATCH, EMB), jnp.float32),
        scratch_types=[
            pltpu.VMEM((b_per_w,), jnp.int32),
            pltpu.VMEM((N_COND, EMB), jnp.float32),
            pltpu.VMEM((b_per_w, EMB), jnp.float32),
        ]
        + [pltpu.SemaphoreType.DMA for _ in range(n_chunks + 3)],
    )
    def run(x_hbm, c_hbm, m_hbm, out_hbm, idx_v, masks_v, xbuf, *sems):
        load_sems = sems[:n_chunks]
        store_sem, idx_sem, msk_sem = sems[n_chunks:]

        wid = lax.axis_index("s") * info.num_cores + lax.axis_index("c")
        base = wid * b_per_w

        idx_cp = pltpu.async_copy(
            c_hbm.at[pl.ds(base, b_per_w)], idx_v, idx_sem)
        msk_cp = pltpu.async_copy(m_hbm, masks_v, msk_sem)
        loads = [
            pltpu.async_copy(
                x_hbm.at[pl.ds(base + j * CHUNK, CHUNK)],
                xbuf.at[pl.ds(j * CHUNK, CHUNK)],
                load_sems[j])
            for j in range(n_chunks)
        ]
        idx_cp.wait()
        msk_cp.wait()

        stores = [
            pltpu.make_async_copy(
                xbuf.at[pl.ds(j * CHUNK, CHUNK)],
                out_hbm.at[pl.ds(base + j * CHUNK, CHUNK)],
                store_sem)
            for j in range(n_chunks)
        ]

        def grp_body(t, carry):
            for j in range(n_chunks):

                @pl.when(t == j * grp_per_chunk)
                def _(j=j):
                    loads[j].wait()
                    if j > 0:
                        stores[j - 1].start()

            cvec = idx_v[pl.ds(t * LANES, LANES)]
            for l in range(LANES):
                r = t * LANES + l
                rowc = cvec[l]
                prods = []
                for g in range(GROUPS):
                    sl = pl.ds(g * LANES, LANES)
                    prods.append(xbuf[r, sl] * masks_v[rowc, sl])
                for g in range(GROUPS):
                    xbuf[r, pl.ds(g * LANES, LANES)] = prods[g]
            return carry

        lax.fori_loop(0, n_chunks * grp_per_chunk, grp_body, 0)
        stores[n_chunks - 1].start()
        for s in stores:
            s.wait()

    return run(embedded_x, c.astype(jnp.int32), masks)


# speedup vs baseline: 1.4763x; 1.0254x over previous
import functools

import jax
import jax.numpy as jnp
from jax import lax
from jax.experimental import pallas as pl
from jax.experimental.pallas import tpu as pltpu
from jax.experimental.pallas import tpu_sc as plsc

BATCH = 16384
EMB = 128
LANES = 16
GROUPS = EMB // LANES
CHUNK = 32
N_COND = 8


def kernel(embedded_x, c, masks):
    info = plsc.get_sparse_core_info()
    n_workers = info.num_cores * info.num_subcores
    b_per_w = BATCH // n_workers
    n_chunks = b_per_w // CHUNK
    grp_per_chunk = CHUNK // LANES

    mesh = plsc.VectorSubcoreMesh(core_axis_name="c", subcore_axis_name="s")

    @functools.partial(
        pl.kernel,
        mesh=mesh,
        out_type=jax.ShapeDtypeStruct((BATCH, EMB), jnp.float32),
        scratch_types=[
            pltpu.VMEM((b_per_w,), jnp.int32),
            pltpu.VMEM((N_COND, EMB), jnp.float32),
            pltpu.VMEM((b_per_w, EMB), jnp.float32),
        ]
        + [pltpu.SemaphoreType.DMA for _ in range(n_chunks + 3)],
    )
    def run(x_hbm, c_hbm, m_hbm, out_hbm, idx_v, masks_v, xbuf, *sems):
        load_sems = sems[:n_chunks]
        store_sem, idx_sem, msk_sem = sems[n_chunks:]

        wid = lax.axis_index("s") * info.num_cores + lax.axis_index("c")
        base = wid * b_per_w

        idx_cp = pltpu.async_copy(
            c_hbm.at[pl.ds(base, b_per_w)], idx_v, idx_sem)
        msk_cp = pltpu.async_copy(m_hbm, masks_v, msk_sem)
        loads = [
            pltpu.async_copy(
                x_hbm.at[pl.ds(base + j * CHUNK, CHUNK)],
                xbuf.at[pl.ds(j * CHUNK, CHUNK)],
                load_sems[j])
            for j in range(n_chunks)
        ]
        idx_cp.wait()
        msk_cp.wait()

        stores = [
            pltpu.make_async_copy(
                xbuf.at[pl.ds(j * CHUNK, CHUNK)],
                out_hbm.at[pl.ds(base + j * CHUNK, CHUNK)],
                store_sem)
            for j in range(n_chunks)
        ]

        def grp_body(t, carry):
            for j in range(n_chunks):

                @pl.when(t == j * grp_per_chunk)
                def _(j=j):
                    loads[j].wait()
                    if j > 0:
                        stores[j - 1].start()

            cvec = idx_v[pl.ds(t * LANES, LANES)]
            for l in range(LANES):
                r = t * LANES + l
                rowc = cvec[l]
                prods = []
                for g in range(GROUPS):
                    sl = pl.ds(g * LANES, LANES)
                    prods.append(xbuf[r, sl] * masks_v[rowc, sl])
                for g in range(GROUPS):
                    xbuf[r, pl.ds(g * LANES, LANES)] = prods[g]
            return carry

        lax.fori_loop(0, n_chunks * grp_per_chunk, grp_body, 0)
        stores[n_chunks - 1].start()
        for s in stores:
            s.wait()

    return run(embedded_x, c.astype(jnp.int32), masks)
